# baseline (speedup 1.0000x reference)
import functools

import jax
import jax.numpy as jnp
from jax import lax
from jax.experimental import pallas as pl
from jax.experimental.pallas import tpu as pltpu
from jax.experimental.pallas import tpu_sc as plsc

N = 10000
NP = 10240
D = 128
E = 320000
NW = 32
EPW = E // NW
B = 80
NB = EPW // B
BLK = 1280
NROWB = NP // BLK
ZR = NP // 16


def _feats_body(xs_ref, xd_ref, w_ref, as_ref, ad_ref, h_ref, als_ref, ald_ref):
    xs = xs_ref[0]
    xd = xd_ref[0]
    w = w_ref[0]
    r = pl.program_id(0)
    a_s = as_ref[pl.ds(r, 1), :]
    a_d = ad_ref[pl.ds(r, 1), :]
    h = jnp.dot(xs, w, preferred_element_type=jnp.float32,
                precision=lax.Precision.HIGHEST)
    hd = jnp.dot(xd, w, preferred_element_type=jnp.float32,
                 precision=lax.Precision.HIGHEST)
    als = jnp.sum(h * a_s, axis=1)
    ald = jnp.sum(hd * a_d, axis=1)
    h_ref[0] = h
    als_ref[0, 0] = als
    ald_ref[0, 0] = ald


def _tc_feats(xs, xd, w, a_s, a_d):
    r = w.shape[0]
    return pl.pallas_call(
        _feats_body,
        grid=(r, NROWB),
        in_specs=[
            pl.BlockSpec((1, BLK, D), lambda i, j: (i, j, 0)),
            pl.BlockSpec((1, BLK, D), lambda i, j: (i, j, 0)),
            pl.BlockSpec((1, D, D), lambda i, j: (i, 0, 0)),
            pl.BlockSpec((r, D), lambda i, j: (0, 0)),
            pl.BlockSpec((r, D), lambda i, j: (0, 0)),
        ],
        out_specs=[
            pl.BlockSpec((1, BLK, D), lambda i, j: (i, j, 0)),
            pl.BlockSpec((1, 1, BLK), lambda i, j: (i, 0, j)),
            pl.BlockSpec((1, 1, BLK), lambda i, j: (i, 0, j)),
        ],
        out_shape=[
            jax.ShapeDtypeStruct((r, NP, D), jnp.float32),
            jax.ShapeDtypeStruct((r, 1, NP), jnp.float32),
            jax.ShapeDtypeStruct((r, 1, NP), jnp.float32),
        ],
    )(xs, xd, w, a_s, a_d)


def _make_sc_edges(nrel):
    mesh = plsc.VectorSubcoreMesh(core_axis_name="c", subcore_axis_name="s")

    @functools.partial(
        pl.kernel,
        out_type=[
            jax.ShapeDtypeStruct((2, nrel, NP, D), jnp.float32),
            jax.ShapeDtypeStruct((2, nrel, NP), jnp.float32),
        ],
        mesh=mesh,
        scratch_types=[
            pltpu.VMEM((NP,), jnp.float32),
            pltpu.VMEM((NP,), jnp.float32),
            pltpu.VMEM((3, B), jnp.int32),
            pltpu.VMEM((3, B), jnp.int32),
            pltpu.VMEM((2, B), jnp.int32),
            pltpu.VMEM((2, B), jnp.float32),
            pltpu.VMEM((2, B, D), jnp.float32),
            pltpu.VMEM((16, D), jnp.float32),
            pltpu.VMEM((ZR,), jnp.float32),
            pltpu.VMEM_SHARED((NP, D), jnp.float32),
            pltpu.VMEM_SHARED((NP,), jnp.float32),
            pltpu.SemaphoreType.DMA,
            pltpu.SemaphoreType.DMA,
        ],
        compiler_params=pltpu.CompilerParams(use_tc_tiling_on_sc=False,
                                             needs_layout_passes=False),
    )
    def sc_edges(h_hbm, als_hbm, ald_hbm, src_hbm, dst_hbm, acc_hbm, den_hbm,
                 als_v, ald_v, sidx_v, didx_v, gidx_v, exv_v, rows_v, zeros_v,
                 zden_v, acc_sh, den_sh, sem_i, sem_g):
        cid = lax.axis_index("c")
        sid = lax.axis_index("s")
        wid = sid * 2 + cid
        zvec = jnp.zeros((16,), jnp.float32)

        def zrow(j, _):
            for kk in range(D // 16):
                zeros_v[j, pl.ds(kk * 16, 16)] = zvec
            return 0

        lax.fori_loop(0, 16, zrow, 0)

        def zden(j, _):
            zden_v[pl.ds(j * 16, 16)] = zvec
            return 0

        lax.fori_loop(0, ZR // 16, zden, 0)

        def load_idx(off, m):
            pltpu.async_copy(src_hbm.at[pl.ds(off, B)], sidx_v.at[m], sem_i)
            pltpu.async_copy(dst_hbm.at[pl.ds(off, B)], didx_v.at[m], sem_i)

        def wait_idx(m):
            pltpu.make_async_copy(src_hbm.at[pl.ds(0, B)], sidx_v.at[m],
                                  sem_i).wait()
            pltpu.make_async_copy(dst_hbm.at[pl.ds(0, B)], didx_v.at[m],
                                  sem_i).wait()

        def compute_w(r, m, p):
            def chunk(j, _):
                s16 = sidx_v[m, pl.ds(j * 16, 16)]
                d16 = didx_v[m, pl.ds(j * 16, 16)]
                a = plsc.load_gather(als_v, [s16])
                b = plsc.load_gather(ald_v, [d16])
                e = a + b
                e = jnp.where(e >= 0.0, e, e * 0.2)
                exv_v[p, pl.ds(j * 16, 16)] = jnp.exp(e)
                gidx_v[p, pl.ds(j * 16, 16)] = s16 + (r * NP)
                return 0

            lax.fori_loop(0, B // 16, chunk, 0)

        def gather_rows(p):
            pltpu.async_copy(h_hbm.at[gidx_v.at[p]], rows_v.at[p], sem_g)

        def wait_gather(p):
            pltpu.make_async_copy(h_hbm.at[pl.ds(0, B)], rows_v.at[p],
                                  sem_g).wait()

        def scale_rows(p):
            def rowblk(j, _):
                ex16 = exv_v[p, pl.ds(j * 16, 16)]
                for jj in range(16):
                    s = ex16[jj]
                    rr = j * 16 + jj
                    for kk in range(D // 16):
                        rows_v[p, rr, pl.ds(kk * 16, 16)] = (
                            rows_v[p, rr, pl.ds(kk * 16, 16)] * s)
                return 0

            lax.fori_loop(0, B // 16, rowblk, 0)

        def scatter_add(m, p):
            pltpu.sync_copy(rows_v.at[p], acc_sh.at[didx_v.at[m]], add=True)
            pltpu.sync_copy(exv_v.at[p], den_sh.at[didx_v.at[m]], add=True)

        for r in range(nrel):
            def zslice(j, _):
                pltpu.sync_copy(zeros_v,
                                acc_sh.at[pl.ds(sid * ZR + j * 16, 16)])
                return 0

            lax.fori_loop(0, ZR // 16, zslice, 0)
            pltpu.sync_copy(zden_v, den_sh.at[pl.ds(sid * ZR, ZR)])

            pltpu.sync_copy(als_hbm.at[r, 0], als_v)
            pltpu.sync_copy(ald_hbm.at[r, 0], ald_v)
            plsc.subcore_barrier()

            base = r * E + wid * EPW

            load_idx(base, 0)
            wait_idx(0)
            compute_w(r, 0, 0)
            gather_rows(0)
            load_idx(base + B, 1)

            def body(i, _):
                @pl.when(i + 1 < NB)
                def _():
                    wait_idx(lax.rem(i + 1, 3))
                    compute_w(r, lax.rem(i + 1, 3), lax.rem(i + 1, 2))
                    gather_rows(lax.rem(i + 1, 2))

                    @pl.when(i + 2 < NB)
                    def _():
                        load_idx(base + (i + 2) * B, lax.rem(i + 2, 3))

                wait_gather(lax.rem(i, 2))
                scale_rows(lax.rem(i, 2))
                scatter_add(lax.rem(i, 3), lax.rem(i, 2))
                return 0

            lax.fori_loop(0, NB, body, 0)
            plsc.subcore_barrier()
            pltpu.sync_copy(acc_sh.at[pl.ds(sid * ZR, ZR)],
                            acc_hbm.at[cid, r, pl.ds(sid * ZR, ZR)])
            pltpu.sync_copy(den_sh.at[pl.ds(sid * ZR, ZR)],
                            den_hbm.at[cid, r, pl.ds(sid * ZR, ZR)])

    return sc_edges


_sc_edges6 = _make_sc_edges(6)
_sc_edges3 = _make_sc_edges(3)


def _combine1_body(acc_ref, den_ref, b_ref, x4_ref):
    a = acc_ref[0] + acc_ref[1]
    den = den_ref[0] + den_ref[1]
    o = a / (den[:, :, None] + 1e-16) + b_ref[:][:, None, :]
    xw = jax.nn.relu((o[3] + o[4] + o[5]) * (1.0 / 3.0))
    x4_ref[0] = xw
    x4_ref[1] = jax.nn.relu(o[0])
    x4_ref[2] = jax.nn.relu(o[1])
    x4_ref[3] = jax.nn.relu(o[2])


def _tc_combine1(acc, den, b):
    return pl.pallas_call(
        _combine1_body,
        grid=(NROWB,),
        in_specs=[
            pl.BlockSpec((2, 6, BLK, D), lambda j: (0, 0, j, 0)),
            pl.BlockSpec((2, 6, BLK), lambda j: (0, 0, j)),
            pl.BlockSpec((6, D), lambda j: (0, 0)),
        ],
        out_specs=pl.BlockSpec((4, BLK, D), lambda j: (0, j, 0)),
        out_shape=jax.ShapeDtypeStruct((4, NP, D), jnp.float32),
    )(acc, den, b)


def _final_body(acc_ref, den_ref, b_ref, fcw_ref, fcb_ref, out_ref):
    a = acc_ref[0] + acc_ref[1]
    den = den_ref[0] + den_ref[1]
    o = a / (den[:, :, None] + 1e-16) + b_ref[:][:, None, :]
    xw = jax.nn.relu((o[0] + o[1] + o[2]) * (1.0 / 3.0))
    out = lax.dot_general(xw, fcw_ref[:], (((1,), (1,)), ((), ())),
                          preferred_element_type=jnp.float32,
                          precision=lax.Precision.HIGHEST)
    out_ref[...] = out + fcb_ref[:]


def _tc_final(acc, den, b, fc_w, fc_b):
    return pl.pallas_call(
        _final_body,
        grid=(NROWB,),
        in_specs=[
            pl.BlockSpec((2, 3, BLK, D), lambda j: (0, 0, j, 0)),
            pl.BlockSpec((2, 3, BLK), lambda j: (0, 0, j)),
            pl.BlockSpec((3, D), lambda j: (0, 0)),
            pl.BlockSpec((D, D), lambda j: (0, 0)),
            pl.BlockSpec((1, D), lambda j: (0, 0)),
        ],
        out_specs=pl.BlockSpec((BLK, D), lambda j: (j, 0)),
        out_shape=jax.ShapeDtypeStruct((NP, D), jnp.float32),
    )(acc, den, b, fc_w, fc_b)


def kernel(x_word, x_dep, x_pos, x_dis, edge_index, W1, att_src1, att_dst1, b1,
           W2, att_src2, att_dst2, b2, fc_W, fc_b):
    pad = ((0, NP - N), (0, 0))
    x4 = jnp.stack([
        jnp.pad(x_word, pad), jnp.pad(x_dep, pad),
        jnp.pad(x_pos, pad), jnp.pad(x_dis, pad),
    ])
    src_sel1 = jnp.array([0, 0, 0, 1, 2, 3])
    dst_sel1 = jnp.array([1, 2, 3, 0, 0, 0])

    src_flat = edge_index[:, 0, :].reshape(-1)
    dst_flat = edge_index[:, 1, :].reshape(-1)

    h1, als1, ald1 = _tc_feats(x4[src_sel1], x4[dst_sel1], W1,
                               att_src1, att_dst1)
    acc1, den1 = _sc_edges6(h1.reshape(6 * NP, D), als1, ald1,
                            src_flat, dst_flat)
    x4b = _tc_combine1(acc1, den1, b1)

    h2, als2, ald2 = _tc_feats(x4b[jnp.array([1, 2, 3])],
                               x4b[jnp.array([0, 0, 0])],
                               W2[3:], att_src2[3:], att_dst2[3:])
    acc2, den2 = _sc_edges3(h2.reshape(3 * NP, D), als2, ald2,
                            src_flat[3 * E:], dst_flat[3 * E:])
    out = _tc_final(acc2, den2, b2[3:], fc_W, fc_b[None, :])
    return out[:N]


# speedup vs baseline: 1.4285x; 1.4285x over previous
import functools

import jax
import jax.numpy as jnp
from jax import lax
from jax.experimental import pallas as pl
from jax.experimental.pallas import tpu as pltpu
from jax.experimental.pallas import tpu_sc as plsc

N = 10000
NP = 10240
D = 128
DW = 144
E = 320000
NW = 32
EPW = E // NW
B = 80
NB = EPW // B
BLK = 1280
NROWB = NP // BLK
ZR = NP // 16


def _feats_body(xs_ref, xd_ref, w_ref, as_ref, ad_ref, h_ref, als_ref, ald_ref):
    xs = xs_ref[0]
    xd = xd_ref[0]
    w = w_ref[0]
    r = pl.program_id(0)
    a_s = as_ref[pl.ds(r, 1), :]
    a_d = ad_ref[pl.ds(r, 1), :]
    h = jnp.dot(xs, w, preferred_element_type=jnp.float32,
                precision=lax.Precision.HIGHEST)
    hd = jnp.dot(xd, w, preferred_element_type=jnp.float32,
                 precision=lax.Precision.HIGHEST)
    als = jnp.sum(h * a_s, axis=1)
    ald = jnp.sum(hd * a_d, axis=1)
    ones = jnp.ones((BLK, 1), jnp.float32)
    pad = jnp.zeros((BLK, DW - D - 1), jnp.float32)
    h_ref[0] = jnp.concatenate([h, ones, pad], axis=1)
    als_ref[0, 0] = als
    ald_ref[0, 0] = ald


def _tc_feats(xs, xd, w, a_s, a_d):
    r = w.shape[0]
    return pl.pallas_call(
        _feats_body,
        grid=(r, NROWB),
        in_specs=[
            pl.BlockSpec((1, BLK, D), lambda i, j: (i, j, 0)),
            pl.BlockSpec((1, BLK, D), lambda i, j: (i, j, 0)),
            pl.BlockSpec((1, D, D), lambda i, j: (i, 0, 0)),
            pl.BlockSpec((r, D), lambda i, j: (0, 0)),
            pl.BlockSpec((r, D), lambda i, j: (0, 0)),
        ],
        out_specs=[
            pl.BlockSpec((1, BLK, DW), lambda i, j: (i, j, 0)),
            pl.BlockSpec((1, 1, BLK), lambda i, j: (i, 0, j)),
            pl.BlockSpec((1, 1, BLK), lambda i, j: (i, 0, j)),
        ],
        out_shape=[
            jax.ShapeDtypeStruct((r, NP, DW), jnp.float32),
            jax.ShapeDtypeStruct((r, 1, NP), jnp.float32),
            jax.ShapeDtypeStruct((r, 1, NP), jnp.float32),
        ],
    )(xs, xd, w, a_s, a_d)


def _make_sc_edges(nrel):
    mesh = plsc.VectorSubcoreMesh(core_axis_name="c", subcore_axis_name="s")

    @functools.partial(
        pl.kernel,
        out_type=jax.ShapeDtypeStruct((2, nrel, NP, DW), jnp.float32),
        mesh=mesh,
        scratch_types=[
            pltpu.VMEM((NP,), jnp.float32),
            pltpu.VMEM((NP,), jnp.float32),
            pltpu.VMEM((2, B), jnp.int32),
            pltpu.VMEM((3, B), jnp.int32),
            pltpu.VMEM((2, B), jnp.int32),
            pltpu.VMEM((2, B), jnp.float32),
            pltpu.VMEM((B, DW), jnp.float32),
            pltpu.VMEM((16, DW), jnp.float32),
            pltpu.VMEM_SHARED((NP, DW), jnp.float32),
            pltpu.SemaphoreType.DMA,
            pltpu.SemaphoreType.DMA,
        ],
        compiler_params=pltpu.CompilerParams(use_tc_tiling_on_sc=False,
                                             needs_layout_passes=False),
    )
    def sc_edges(h_hbm, als_hbm, ald_hbm, src_hbm, dst_hbm, acc_hbm,
                 als_v, ald_v, sidx_v, didx_v, gidx_v, exv_v, rows_v, zeros_v,
                 acc_sh, sem_i, sem_g):
        cid = lax.axis_index("c")
        sid = lax.axis_index("s")
        wid = sid * 2 + cid
        zvec = jnp.zeros((16,), jnp.float32)

        def zrow(j, _):
            for kk in range(DW // 16):
                zeros_v[j, pl.ds(kk * 16, 16)] = zvec
            return 0

        lax.fori_loop(0, 16, zrow, 0)

        def load_idx(off, m2, m3):
            pltpu.async_copy(src_hbm.at[pl.ds(off, B)], sidx_v.at[m2], sem_i)
            pltpu.async_copy(dst_hbm.at[pl.ds(off, B)], didx_v.at[m3], sem_i)

        def wait_idx(m2, m3):
            pltpu.make_async_copy(src_hbm.at[pl.ds(0, B)], sidx_v.at[m2],
                                  sem_i).wait()
            pltpu.make_async_copy(dst_hbm.at[pl.ds(0, B)], didx_v.at[m3],
                                  sem_i).wait()

        def compute_w(r, m2, m3, p):
            def chunk(j, _):
                s16 = sidx_v[m2, pl.ds(j * 16, 16)]
                d16 = didx_v[m3, pl.ds(j * 16, 16)]
                a = plsc.load_gather(als_v, [s16])
                b = plsc.load_gather(ald_v, [d16])
                e = a + b
                e = jnp.where(e >= 0.0, e, e * 0.2)
                exv_v[p, pl.ds(j * 16, 16)] = jnp.exp(e)
                gidx_v[p, pl.ds(j * 16, 16)] = s16 + (r * NP)
                return 0

            lax.fori_loop(0, B // 16, chunk, 0)

        def gather_rows(p):
            pltpu.async_copy(h_hbm.at[gidx_v.at[p]], rows_v, sem_g)

        def wait_gather():
            pltpu.make_async_copy(h_hbm.at[pl.ds(0, B)], rows_v,
                                  sem_g).wait()

        def scale_rows(p):
            def rowblk(j, _):
                ex16 = exv_v[p, pl.ds(j * 16, 16)]
                for jj in range(16):
                    s = ex16[jj]
                    rr = j * 16 + jj
                    for kk in range(DW // 16):
                        rows_v[rr, pl.ds(kk * 16, 16)] = (
                            rows_v[rr, pl.ds(kk * 16, 16)] * s)
                return 0

            lax.fori_loop(0, B // 16, rowblk, 0)

        def scatter_add(m3):
            pltpu.sync_copy(rows_v, acc_sh.at[didx_v.at[m3]], add=True)

        for r in range(nrel):
            def zslice(j, _):
                pltpu.sync_copy(zeros_v,
                                acc_sh.at[pl.ds(sid * ZR + j * 16, 16)])
                return 0

            lax.fori_loop(0, ZR // 16, zslice, 0)

            pltpu.sync_copy(als_hbm.at[r, 0], als_v)
            pltpu.sync_copy(ald_hbm.at[r, 0], ald_v)
            plsc.subcore_barrier()

            base = r * E + wid * EPW

            load_idx(base, 0, 0)
            wait_idx(0, 0)
            compute_w(r, 0, 0, 0)
            load_idx(base + B, 1, 1)

            def body(i, _):
                gather_rows(lax.rem(i, 2))

                @pl.when(i + 1 < NB)
                def _():
                    wait_idx(lax.rem(i + 1, 2), lax.rem(i + 1, 3))
                    compute_w(r, lax.rem(i + 1, 2), lax.rem(i + 1, 3),
                              lax.rem(i + 1, 2))

                    @pl.when(i + 2 < NB)
                    def _():
                        load_idx(base + (i + 2) * B, lax.rem(i + 2, 2),
                                 lax.rem(i + 2, 3))

                wait_gather()
                scale_rows(lax.rem(i, 2))
                scatter_add(lax.rem(i, 3))
                return 0

            lax.fori_loop(0, NB, body, 0)
            plsc.subcore_barrier()
            pltpu.sync_copy(acc_sh.at[pl.ds(sid * ZR, ZR)],
                            acc_hbm.at[cid, r, pl.ds(sid * ZR, ZR)])

    return sc_edges


_sc_edges6 = _make_sc_edges(6)
_sc_edges3 = _make_sc_edges(3)


def _combine1_body(acc_ref, b_ref, x4_ref):
    a = acc_ref[0] + acc_ref[1]
    num = a[:, :, :D]
    den = a[:, :, D:D + 1]
    o = num / (den + 1e-16) + b_ref[:][:, None, :]
    xw = jax.nn.relu((o[3] + o[4] + o[5]) * (1.0 / 3.0))
    x4_ref[0] = xw
    x4_ref[1] = jax.nn.relu(o[0])
    x4_ref[2] = jax.nn.relu(o[1])
    x4_ref[3] = jax.nn.relu(o[2])


def _tc_combine1(acc, b):
    return pl.pallas_call(
        _combine1_body,
        grid=(NROWB,),
        in_specs=[
            pl.BlockSpec((2, 6, BLK, DW), lambda j: (0, 0, j, 0)),
            pl.BlockSpec((6, D), lambda j: (0, 0)),
        ],
        out_specs=pl.BlockSpec((4, BLK, D), lambda j: (0, j, 0)),
        out_shape=jax.ShapeDtypeStruct((4, NP, D), jnp.float32),
    )(acc, b)


def _final_body(acc_ref, b_ref, fcw_ref, fcb_ref, out_ref):
    a = acc_ref[0] + acc_ref[1]
    num = a[:, :, :D]
    den = a[:, :, D:D + 1]
    o = num / (den + 1e-16) + b_ref[:][:, None, :]
    xw = jax.nn.relu((o[0] + o[1] + o[2]) * (1.0 / 3.0))
    out = lax.dot_general(xw, fcw_ref[:], (((1,), (1,)), ((), ())),
                          preferred_element_type=jnp.float32,
                          precision=lax.Precision.HIGHEST)
    out_ref[...] = out + fcb_ref[:]


def _tc_final(acc, b, fc_w, fc_b):
    return pl.pallas_call(
        _final_body,
        grid=(NROWB,),
        in_specs=[
            pl.BlockSpec((2, 3, BLK, DW), lambda j: (0, 0, j, 0)),
            pl.BlockSpec((3, D), lambda j: (0, 0)),
            pl.BlockSpec((D, D), lambda j: (0, 0)),
            pl.BlockSpec((1, D), lambda j: (0, 0)),
        ],
        out_specs=pl.BlockSpec((BLK, D), lambda j: (j, 0)),
        out_shape=jax.ShapeDtypeStruct((NP, D), jnp.float32),
    )(acc, b, fc_w, fc_b)


def kernel(x_word, x_dep, x_pos, x_dis, edge_index, W1, att_src1, att_dst1, b1,
           W2, att_src2, att_dst2, b2, fc_W, fc_b):
    pad = ((0, NP - N), (0, 0))
    x4 = jnp.stack([
        jnp.pad(x_word, pad), jnp.pad(x_dep, pad),
        jnp.pad(x_pos, pad), jnp.pad(x_dis, pad),
    ])
    src_sel1 = jnp.array([0, 0, 0, 1, 2, 3])
    dst_sel1 = jnp.array([1, 2, 3, 0, 0, 0])

    src_flat = edge_index[:, 0, :].reshape(-1)
    dst_flat = edge_index[:, 1, :].reshape(-1)

    h1, als1, ald1 = _tc_feats(x4[src_sel1], x4[dst_sel1], W1,
                               att_src1, att_dst1)
    acc1 = _sc_edges6(h1.reshape(6 * NP, DW), als1, ald1, src_flat, dst_flat)
    x4b = _tc_combine1(acc1, b1)

    h2, als2, ald2 = _tc_feats(x4b[jnp.array([1, 2, 3])],
                               x4b[jnp.array([0, 0, 0])],
                               W2[3:], att_src2[3:], att_dst2[3:])
    acc2 = _sc_edges3(h2.reshape(3 * NP, DW), als2, ald2,
                      src_flat[3 * E:], dst_flat[3 * E:])
    out = _tc_final(acc2, b2[3:], fc_W, fc_b[None, :])
    return out[:N]
